# Initial kernel scaffold; baseline (speedup 1.0000x reference)
import jax
import jax.numpy as jnp
from jax.experimental import pallas as pl


def kernel(detection_result, gt_grid):
    raise NotImplementedError("write your pallas kernel here")


# speedup vs baseline: 8.9447x; 8.9447x over previous
import functools

import jax
import jax.numpy as jnp
from jax.experimental import pallas as pl
from jax.experimental.pallas import tpu as pltpu

_B = 1024
_NBOX = 5
_PER = 25
_HW = 169
_BB = 32
_LAMBDA_COORD = 5.0
_LAMBDA_NOOBJ = 0.5


def _loss_body(det_ref, gt_ref, loss_ref, obj_ref, noobj_ref, conf_ref, acc):
    step = pl.program_id(0)
    nsteps = pl.num_programs(0)

    @pl.when(step == 0)
    def _init():
        for i in range(16):
            acc[i] = 0.0

    d = det_ref[...]
    g = gt_ref[...]
    err = (d - g) ** 2
    obj = g[:, :, 4, :]
    mf = (obj == 1.0).astype(jnp.float32)

    coord_part = jnp.sum(err[:, :, 0:4, :] * mf[:, :, None, :])
    conf_e = err[:, :, 4, :]
    conf_obj_part = jnp.sum(conf_e * mf)
    conf_all_part = jnp.sum(conf_e)
    cnt_part = jnp.sum(mf)

    acc[0] += coord_part
    acc[1] += conf_obj_part
    acc[2] += conf_all_part
    acc[3] += cnt_part
    for b in range(_NBOX):
        acc[4 + b] += jnp.sum(err[:, b, 5:25, :] * mf[:, b, None, :])
        acc[9 + b] += jnp.sum(mf[:, b, :])

    @pl.when(step == nsteps - 1)
    def _finish():
        cnt = acc[3]
        total = float(_B * _NBOX * _HW)
        coord = jnp.where(cnt > 0, acc[0] / cnt, 0.0)
        conf_obj = jnp.where(cnt > 0, acc[1] / cnt, 0.0)
        obj_loss = _LAMBDA_COORD * coord + conf_obj
        noobj_cnt = total - cnt
        no_obj_loss = _LAMBDA_NOOBJ * jnp.where(
            noobj_cnt > 0, (acc[2] - acc[1]) / noobj_cnt, 0.0
        )
        confidence = 0.0
        for b in range(_NBOX):
            cnt_b = acc[9 + b] * 20.0
            confidence = confidence + jnp.where(cnt_b > 0, acc[4 + b] / cnt_b, 0.0)
        loss_ref[0, 0] = obj_loss + no_obj_loss + confidence
        obj_ref[0, 0] = obj_loss
        noobj_ref[0, 0] = no_obj_loss
        conf_ref[0, 0] = confidence


@jax.jit
def _detection_loss(det, gt):
    det4 = det.reshape(_B, _NBOX, _PER, _HW)
    gt4 = gt.reshape(_B, _NBOX, _PER, _HW)
    grid = (_B // _BB,)
    in_spec = pl.BlockSpec(
        (_BB, _NBOX, _PER, _HW), lambda i: (i, 0, 0, 0)
    )
    out_spec = pl.BlockSpec(memory_space=pltpu.SMEM)
    scalar = jax.ShapeDtypeStruct((1, 1), jnp.float32)
    outs = pl.pallas_call(
        _loss_body,
        grid=grid,
        in_specs=[in_spec, in_spec],
        out_specs=[out_spec] * 4,
        out_shape=[scalar] * 4,
        scratch_shapes=[pltpu.SMEM((16,), jnp.float32)],
    )(det4, gt4)
    loss, obj_loss, no_obj_loss, confidence = [o[0, 0] for o in outs]
    return (loss, obj_loss, no_obj_loss, confidence)


def kernel(detection_result, gt_grid):
    return _detection_loss(detection_result, gt_grid)
